# Initial kernel scaffold; baseline (speedup 1.0000x reference)
import jax
import jax.numpy as jnp
from jax.experimental import pallas as pl


def kernel(x, y, W1, b1, gamma, beta, W2, b2):
    raise NotImplementedError("write your pallas kernel here")


# speedup vs baseline: 107.5871x; 107.5871x over previous
import functools

import jax
import jax.numpy as jnp
from jax import lax
from jax.experimental import pallas as pl
from jax.experimental.pallas import tpu as pltpu
from jax.experimental.pallas import tpu_sc as plsc

_ZNUM = 4096
_HID = 16


def _mlp_body(p_ref, q_ref, x_ref, y_ref, idx_ref, val_ref):
    dz = q_ref[0]
    hdz = q_ref[1]
    b2s = q_ref[2]
    nb = x_ref.shape[1]

    def step(i, carry):
        sl = pl.ds(i * 128, 128)
        xv = x_ref[:, sl]
        yv = y_ref[:, sl]
        t = (xv - hdz) / dz
        idi = jnp.clip(jnp.ceil(t).astype(jnp.int32), 0, _ZNUM - 1)
        idx_ref[:, sl] = idi
        zv = idi.astype(jnp.float32) * dz
        hs = []
        ssq = None
        for k in range(_HID):
            hk = xv * p_ref[0, k] + zv * p_ref[1, k] + yv * p_ref[2, k] + p_ref[3, k]
            hs.append(hk)
            ssq = hk * hk if ssq is None else ssq + hk * hk
        rstd = lax.rsqrt(ssq * (1.0 / _HID) + 1e-5)
        out = None
        for k in range(_HID):
            g = hs[k] * rstd * p_ref[4, k] + p_ref[5, k]
            a = jax.nn.gelu(g, approximate=True)
            out = a * p_ref[6, k] if out is None else out + a * p_ref[6, k]
        val_ref[:, sl] = (out + b2s) * yv
        return carry

    lax.fori_loop(0, nb // 128, step, 0)


def _tc_mlp(x, y2, P, Q):
    b, n = x.shape
    bb, nb = 8, 4096
    grid = (b // bb, n // nb)
    blk = pl.BlockSpec((bb, nb), lambda i, j: (i, j))
    return pl.pallas_call(
        _mlp_body,
        grid=grid,
        in_specs=[
            pl.BlockSpec(memory_space=pltpu.SMEM),
            pl.BlockSpec(memory_space=pltpu.SMEM),
            blk,
            blk,
        ],
        out_specs=[blk, blk],
        out_shape=[
            jax.ShapeDtypeStruct((b, n), jnp.int32),
            jax.ShapeDtypeStruct((b, n), jnp.float32),
        ],
    )(P, Q, x, y2)


_CHUNK = 16384


def _make_hist(b, n):
    info = plsc.get_sparse_core_info()
    nw = info.num_cores * info.num_subcores
    rpw = b // nw
    mesh = plsc.VectorSubcoreMesh(core_axis_name="c", subcore_axis_name="s")

    @functools.partial(
        pl.kernel,
        mesh=mesh,
        compiler_params=pltpu.CompilerParams(needs_layout_passes=False),
        out_type=jax.ShapeDtypeStruct((b, _ZNUM), jnp.float32),
        scratch_types=[
            pltpu.VMEM((_CHUNK,), jnp.int32),
            pltpu.VMEM((_CHUNK,), jnp.float32),
            pltpu.VMEM((_ZNUM,), jnp.float32),
            pltpu.VMEM((_ZNUM,), jnp.float32),
            pltpu.VMEM((_ZNUM,), jnp.float32),
        ],
    )
    def hist(idx_hbm, val_hbm, out_hbm, idx_v, val_v, sums_v, cnts_v, outb_v):
        wid = lax.axis_index("s") * info.num_cores + lax.axis_index("c")
        zero16 = jnp.zeros((16,), jnp.float32)
        ones16 = jnp.full((16,), 1.0, jnp.float32)

        for r in range(rpw):
            row = wid * rpw + r

            def zbody(i, carry):
                sums_v[pl.ds(i * 16, 16)] = zero16
                cnts_v[pl.ds(i * 16, 16)] = zero16
                return carry

            lax.fori_loop(0, _ZNUM // 16, zbody, 0)

            for ci in range(n // _CHUNK):
                pltpu.sync_copy(idx_hbm.at[row, pl.ds(ci * _CHUNK, _CHUNK)], idx_v)
                pltpu.sync_copy(val_hbm.at[row, pl.ds(ci * _CHUNK, _CHUNK)], val_v)

                def ibody(j, carry):
                    sl = pl.ds(j * 16, 16)
                    binv = idx_v[sl]
                    plsc.addupdate_scatter(sums_v, [binv], val_v[sl])
                    plsc.addupdate_scatter(cnts_v, [binv], ones16)
                    return carry

                lax.fori_loop(0, _CHUNK // 16, ibody, 0)

            def mbody(i, carry):
                sl = pl.ds(i * 16, 16)
                outb_v[sl] = sums_v[sl] / jnp.maximum(cnts_v[sl], 1.0)
                return carry

            lax.fori_loop(0, _ZNUM // 16, mbody, 0)
            pltpu.sync_copy(outb_v, out_hbm.at[row])

    return hist


def kernel(x, y, W1, b1, gamma, beta, W2, b2):
    b, n = x.shape
    y2 = y[..., 0]
    zgrid = jnp.linspace(0.0, 1.0, _ZNUM).astype(jnp.float32)
    dz = zgrid[1] - zgrid[0]
    wx, wz, wy = W1[0], W1[1], W1[2]
    P = jnp.stack(
        [
            wx - jnp.mean(wx),
            wz - jnp.mean(wz),
            wy - jnp.mean(wy),
            b1 - jnp.mean(b1),
            gamma,
            beta,
            W2[:, 0],
            jnp.zeros((_HID,), jnp.float32),
        ]
    )
    Q = jnp.stack([dz, dz * 0.5, b2[0], jnp.float32(0.0)])
    idx, vals = _tc_mlp(x, y2, P, Q)
    mean = _make_hist(b, n)(idx, vals)
    return mean[:, None, :]


# speedup vs baseline: 114.7217x; 1.0663x over previous
import functools

import jax
import jax.numpy as jnp
from jax import lax
from jax.experimental import pallas as pl
from jax.experimental.pallas import tpu as pltpu
from jax.experimental.pallas import tpu_sc as plsc

_ZNUM = 4096
_HID = 16


def _tree_sum(xs):
    xs = list(xs)
    while len(xs) > 1:
        nxt = [xs[i] + xs[i + 1] for i in range(0, len(xs) - 1, 2)]
        if len(xs) % 2:
            nxt.append(xs[-1])
        xs = nxt
    return xs[0]


def _mlp_body(p_ref, q_ref, x_ref, y_ref, idx_ref, val_ref):
    dz = q_ref[0]
    hdz = q_ref[1]
    b2s = q_ref[2]
    nb = x_ref.shape[1]
    pa = [p_ref[0, k] for k in range(_HID)]
    pb = [p_ref[1, k] for k in range(_HID)]
    pc = [p_ref[2, k] for k in range(_HID)]
    pd = [p_ref[3, k] for k in range(_HID)]
    pg = [p_ref[4, k] for k in range(_HID)]
    pbe = [p_ref[5, k] for k in range(_HID)]
    pw2 = [p_ref[6, k] for k in range(_HID)]
    lanes = 256

    def step(i, carry):
        sl = pl.ds(i * lanes, lanes)
        xv = x_ref[:, sl]
        yv = y_ref[:, sl]
        t = (xv - hdz) / dz
        idi = jnp.clip(jnp.ceil(t).astype(jnp.int32), 0, _ZNUM - 1)
        idx_ref[:, sl] = idi
        zv = idi.astype(jnp.float32) * dz
        hs = [xv * pa[k] + zv * pb[k] + yv * pc[k] + pd[k] for k in range(_HID)]
        ssq = _tree_sum([h * h for h in hs])
        rstd = lax.rsqrt(ssq * (1.0 / _HID) + 1e-5)
        outs = []
        for k in range(_HID):
            g = hs[k] * rstd * pg[k] + pbe[k]
            outs.append(jax.nn.gelu(g, approximate=True) * pw2[k])
        val_ref[:, sl] = (_tree_sum(outs) + b2s) * yv
        return carry

    lax.fori_loop(0, nb // lanes, step, 0)


def _tc_mlp(x, y2, P, Q):
    b, n = x.shape
    bb, nb = 8, 4096
    grid = (b // bb, n // nb)
    blk = pl.BlockSpec((bb, nb), lambda i, j: (i, j))
    return pl.pallas_call(
        _mlp_body,
        grid=grid,
        in_specs=[
            pl.BlockSpec(memory_space=pltpu.SMEM),
            pl.BlockSpec(memory_space=pltpu.SMEM),
            blk,
            blk,
        ],
        out_specs=[blk, blk],
        out_shape=[
            jax.ShapeDtypeStruct((b, n), jnp.int32),
            jax.ShapeDtypeStruct((b, n), jnp.float32),
        ],
    )(P, Q, x, y2)


_CHUNK = 16384


def _make_hist(b, n):
    info = plsc.get_sparse_core_info()
    nw = info.num_cores * info.num_subcores
    rpw = b // nw
    mesh = plsc.VectorSubcoreMesh(core_axis_name="c", subcore_axis_name="s")

    @functools.partial(
        pl.kernel,
        mesh=mesh,
        compiler_params=pltpu.CompilerParams(needs_layout_passes=False),
        out_type=jax.ShapeDtypeStruct((b, _ZNUM), jnp.float32),
        scratch_types=[
            pltpu.VMEM((_CHUNK,), jnp.int32),
            pltpu.VMEM((_CHUNK,), jnp.float32),
            pltpu.VMEM((_ZNUM,), jnp.float32),
            pltpu.VMEM((_ZNUM,), jnp.float32),
            pltpu.VMEM((_ZNUM,), jnp.float32),
        ],
    )
    def hist(idx_hbm, val_hbm, out_hbm, idx_v, val_v, sums_v, cnts_v, outb_v):
        wid = lax.axis_index("s") * info.num_cores + lax.axis_index("c")
        zero16 = jnp.zeros((16,), jnp.float32)
        ones16 = jnp.full((16,), 1.0, jnp.float32)

        for r in range(rpw):
            row = wid * rpw + r

            def zbody(i, carry):
                sums_v[pl.ds(i * 16, 16)] = zero16
                cnts_v[pl.ds(i * 16, 16)] = zero16
                return carry

            lax.fori_loop(0, _ZNUM // 16, zbody, 0)

            for ci in range(n // _CHUNK):
                pltpu.sync_copy(idx_hbm.at[row, pl.ds(ci * _CHUNK, _CHUNK)], idx_v)
                pltpu.sync_copy(val_hbm.at[row, pl.ds(ci * _CHUNK, _CHUNK)], val_v)

                def ibody(j, carry):
                    sl = pl.ds(j * 16, 16)
                    binv = idx_v[sl]
                    plsc.addupdate_scatter(sums_v, [binv], val_v[sl])
                    plsc.addupdate_scatter(cnts_v, [binv], ones16)
                    return carry

                lax.fori_loop(0, _CHUNK // 16, ibody, 0)

            def mbody(i, carry):
                sl = pl.ds(i * 16, 16)
                outb_v[sl] = sums_v[sl] / jnp.maximum(cnts_v[sl], 1.0)
                return carry

            lax.fori_loop(0, _ZNUM // 16, mbody, 0)
            pltpu.sync_copy(outb_v, out_hbm.at[row])

    return hist


def kernel(x, y, W1, b1, gamma, beta, W2, b2):
    b, n = x.shape
    y2 = y[..., 0]
    zgrid = jnp.linspace(0.0, 1.0, _ZNUM).astype(jnp.float32)
    dz = zgrid[1] - zgrid[0]
    wx, wz, wy = W1[0], W1[1], W1[2]
    P = jnp.stack(
        [
            wx - jnp.mean(wx),
            wz - jnp.mean(wz),
            wy - jnp.mean(wy),
            b1 - jnp.mean(b1),
            gamma,
            beta,
            W2[:, 0],
            jnp.zeros((_HID,), jnp.float32),
        ]
    )
    Q = jnp.stack([dz, dz * 0.5, b2[0], jnp.float32(0.0)])
    idx, vals = _tc_mlp(x, y2, P, Q)
    mean = _make_hist(b, n)(idx, vals)
    return mean[:, None, :]
